# baseline (speedup 1.0000x reference)
import jax
import jax.numpy as jnp
from jax import lax
from jax.experimental import pallas as pl
from jax.experimental.pallas import tpu as pltpu
from jax.experimental.pallas import tpu_sc as plsc

_BATCH = 4096
_FIELDS = 26
_LANES = 16
_NUM_CORES = 2
_NUM_SUBCORES = 16
_NUM_WORKERS = _NUM_CORES * _NUM_SUBCORES
_BPW = _BATCH // _NUM_WORKERS
_CHUNKS = _BPW // _LANES
_SLAB = _FIELDS * _BPW


def _wid():
    return lax.axis_index("s") * _NUM_CORES + lax.axis_index("c")


def _body_idx(xt_hbm, off_hbm, bias_hbm, idx_hbm, bias_out_hbm,
              xv, off_v, bias_v, idx1d, sem_x):
    w = _wid()
    base = w * _BPW

    cpx = pltpu.make_async_copy(xt_hbm.at[:, pl.ds(base, _BPW)], xv, sem_x)
    cpx.start()
    pltpu.sync_copy(off_hbm, off_v)
    cpx.wait()

    for f in range(_FIELDS):
        off_b = off_v[f, :]
        for c in range(_CHUNKS):
            idx1d[pl.ds(f * _BPW + c * _LANES, _LANES)] = (
                xv[f, pl.ds(c * _LANES, _LANES)] + off_b)

    pltpu.sync_copy(idx1d, idx_hbm.at[w])
    @pl.when(w == 0)
    def _():
        pltpu.sync_copy(bias_hbm, bias_v)
        pltpu.sync_copy(bias_v, bias_out_hbm)


def _body_gather(idx_hbm, bias_hbm, table_hbm, out_hbm,
                 idx1d, val1d, bias_v, acc_v, sem_g):
    w = _wid()
    base = w * _BPW

    pltpu.sync_copy(idx_hbm.at[w], idx1d)
    pltpu.sync_copy(bias_hbm, bias_v)

    cp = pltpu.make_async_copy(table_hbm.at[idx1d], val1d, sem_g)
    cp.start()
    cp.wait()

    bias_vec = bias_v[...]
    accs = [bias_vec] * _CHUNKS
    for f in range(_FIELDS):
        for c in range(_CHUNKS):
            accs[c] = accs[c] + val1d[pl.ds(f * _BPW + c * _LANES, _LANES)]
    for c in range(_CHUNKS):
        acc_v[pl.ds(c * _LANES, _LANES)] = accs[c]

    pltpu.sync_copy(acc_v, out_hbm.at[pl.ds(base, _BPW)])


@jax.jit
def kernel(x, table, bias, offsets):
    bias_b = jnp.broadcast_to(bias.astype(jnp.float32), (_LANES,))
    off_b2d = jnp.broadcast_to(
        offsets.astype(jnp.int32)[:, None], (_FIELDS, _LANES))
    x_t = x.T
    table_flat = table.reshape(-1)
    mesh = plsc.VectorSubcoreMesh(core_axis_name="c", subcore_axis_name="s",
                                  num_cores=_NUM_CORES,
                                  num_subcores=_NUM_SUBCORES)
    params = pltpu.CompilerParams(needs_layout_passes=False)

    run_idx = pl.kernel(
        _body_idx,
        out_type=(
            jax.ShapeDtypeStruct((_NUM_WORKERS, _SLAB), jnp.int32),
            jax.ShapeDtypeStruct((_LANES,), jnp.float32),
        ),
        mesh=mesh,
        compiler_params=params,
        scratch_types=[
            pltpu.VMEM((_FIELDS, _BPW), jnp.int32),
            pltpu.VMEM((_FIELDS, _LANES), jnp.int32),
            pltpu.VMEM((_LANES,), jnp.float32),
            pltpu.VMEM((_SLAB,), jnp.int32),
            pltpu.SemaphoreType.DMA,
        ],
    )
    idx_all, bias_vec = run_idx(x_t, off_b2d, bias_b)

    run_gather = pl.kernel(
        _body_gather,
        out_type=jax.ShapeDtypeStruct((_BATCH,), jnp.float32),
        mesh=mesh,
        compiler_params=params,
        scratch_types=[
            pltpu.VMEM((_SLAB,), jnp.int32),
            pltpu.VMEM((_SLAB,), jnp.float32),
            pltpu.VMEM((_LANES,), jnp.float32),
            pltpu.VMEM((_BPW,), jnp.float32),
            pltpu.SemaphoreType.DMA,
        ],
    )
    out = run_gather(idx_all, bias_vec, table_flat)
    return out.reshape(_BATCH, 1)


# speedup vs baseline: 1.0011x; 1.0011x over previous
import jax
import jax.numpy as jnp
from jax import lax
from jax.experimental import pallas as pl
from jax.experimental.pallas import tpu as pltpu
from jax.experimental.pallas import tpu_sc as plsc

_BATCH = 4096
_FIELDS = 26
_LANES = 16
_NUM_CORES = 2
_NUM_SUBCORES = 16
_NUM_WORKERS = _NUM_CORES * _NUM_SUBCORES
_BPW = _BATCH // _NUM_WORKERS
_CHUNKS = _BPW // _LANES
_SLAB = _FIELDS * _BPW


def _wid():
    return lax.axis_index("s") * _NUM_CORES + lax.axis_index("c")


def _body_idx(xt_hbm, off_hbm, idx_hbm, xv, off_v, idx1d, sem_x):
    w = _wid()
    base = w * _BPW

    cpx = pltpu.make_async_copy(xt_hbm.at[:, pl.ds(base, _BPW)], xv, sem_x)
    cpx.start()
    pltpu.sync_copy(off_hbm, off_v)
    cpx.wait()

    for f in range(_FIELDS):
        off_b = off_v[f, :]
        for c in range(_CHUNKS):
            idx1d[pl.ds(f * _BPW + c * _LANES, _LANES)] = (
                xv[f, pl.ds(c * _LANES, _LANES)] + off_b)

    pltpu.sync_copy(idx1d, idx_hbm.at[w])


def _body_gather(idx_hbm, bias_hbm, table_hbm, out_hbm,
                 idx1d, val1d, bias_v, acc_v, sem_g):
    w = _wid()
    base = w * _BPW

    pltpu.sync_copy(idx_hbm.at[w], idx1d)
    pltpu.sync_copy(bias_hbm, bias_v)

    cp = pltpu.make_async_copy(table_hbm.at[idx1d], val1d, sem_g)
    cp.start()
    cp.wait()

    bias_vec = bias_v[...]
    accs = [bias_vec] * _CHUNKS
    for f in range(_FIELDS):
        for c in range(_CHUNKS):
            accs[c] = accs[c] + val1d[pl.ds(f * _BPW + c * _LANES, _LANES)]
    for c in range(_CHUNKS):
        acc_v[pl.ds(c * _LANES, _LANES)] = accs[c]

    pltpu.sync_copy(acc_v, out_hbm.at[pl.ds(base, _BPW)])


@jax.jit
def kernel(x, table, bias, offsets):
    bias_b = jnp.broadcast_to(bias.astype(jnp.float32), (_LANES,))
    off_b2d = jnp.broadcast_to(
        offsets.astype(jnp.int32)[:, None], (_FIELDS, _LANES))
    x_t = x.T
    table_flat = table.reshape(-1)
    mesh = plsc.VectorSubcoreMesh(core_axis_name="c", subcore_axis_name="s",
                                  num_cores=_NUM_CORES,
                                  num_subcores=_NUM_SUBCORES)
    params = pltpu.CompilerParams(needs_layout_passes=False)

    run_idx = pl.kernel(
        _body_idx,
        out_type=jax.ShapeDtypeStruct((_NUM_WORKERS, _SLAB), jnp.int32),
        mesh=mesh,
        compiler_params=params,
        scratch_types=[
            pltpu.VMEM((_FIELDS, _BPW), jnp.int32),
            pltpu.VMEM((_FIELDS, _LANES), jnp.int32),
            pltpu.VMEM((_SLAB,), jnp.int32),
            pltpu.SemaphoreType.DMA,
        ],
    )
    idx_all = run_idx(x_t, off_b2d)

    run_gather = pl.kernel(
        _body_gather,
        out_type=jax.ShapeDtypeStruct((_BATCH,), jnp.float32),
        mesh=mesh,
        compiler_params=params,
        scratch_types=[
            pltpu.VMEM((_SLAB,), jnp.int32),
            pltpu.VMEM((_SLAB,), jnp.float32),
            pltpu.VMEM((_LANES,), jnp.float32),
            pltpu.VMEM((_BPW,), jnp.float32),
            pltpu.SemaphoreType.DMA,
        ],
    )
    out = run_gather(idx_all, bias_b, table_flat)
    return out.reshape(_BATCH, 1)
